# baseline (speedup 1.0000x reference)
import functools

import jax
import jax.numpy as jnp
from jax.experimental import pallas as pl
from jax.experimental.pallas import tpu as pltpu

ALPHA = 0.2
NEG = -9e15

_INTERPRET = False


def _divisor_block(n, target):
    best = None
    for b in range(8, min(n, target) + 1, 8):
        if n % b == 0:
            best = b
    return best if best is not None else n


def _leaky_relu(v):
    return jnp.maximum(v, ALPHA * v)


def _elu(v):
    return jnp.where(v > 0, v, jnp.exp(jnp.minimum(v, 0.0)) - 1.0)


def _attend(mask, s, dt, wh):
    t = _leaky_relu(s + dt)
    e = jnp.where(mask, t, NEG)
    m = jnp.max(e, axis=1, keepdims=True)
    p = jnp.exp(e - m)
    l = jnp.sum(p, axis=1, keepdims=True)
    acc = jnp.dot(p, wh, preferred_element_type=jnp.float32)
    return acc / l


def _proj_body(x_ref, w0_ref, a0_ref, w1_ref, a1_ref,
               wh0_ref, s0_ref, d0_ref, wh1_ref, s1_ref, d1_ref):
    x = x_ref[...]
    d_hid = w0_ref.shape[1]
    for w_ref, a_ref, wh_ref, s_ref, d_ref in (
        (w0_ref, a0_ref, wh0_ref, s0_ref, d0_ref),
        (w1_ref, a1_ref, wh1_ref, s1_ref, d1_ref),
    ):
        wh = jnp.dot(x, w_ref[...], preferred_element_type=jnp.float32)
        wh_ref[...] = wh
        s_ref[...] = jnp.dot(wh, a_ref[:d_hid], preferred_element_type=jnp.float32)
        d_ref[...] = jnp.dot(wh, a_ref[d_hid:], preferred_element_type=jnp.float32)


def _projections(x, w0, a0, w1, a1):
    n, nfeat = x.shape
    d_hid = w0.shape[1]
    br = _divisor_block(n, 2500)
    grid = (n // br,)
    out_shapes = []
    for _ in range(2):
        out_shapes += [
            jax.ShapeDtypeStruct((n, d_hid), jnp.float32),
            jax.ShapeDtypeStruct((n, 1), jnp.float32),
            jax.ShapeDtypeStruct((n, 1), jnp.float32),
        ]
    full = lambda shape: pl.BlockSpec(shape, lambda i: (0, 0))
    row = lambda width: pl.BlockSpec((br, width), lambda i: (i, 0))
    return pl.pallas_call(
        _proj_body,
        grid=grid,
        in_specs=[
            row(nfeat),
            full(w0.shape), full(a0.shape),
            full(w1.shape), full(a1.shape),
        ],
        out_specs=[row(d_hid), row(1), row(1)] * 2,
        out_shape=out_shapes,
        compiler_params=pltpu.CompilerParams(dimension_semantics=("parallel",)),
        interpret=_INTERPRET,
    )(x, w0, a0, w1, a1)


def _flash12_body(adj_ref, s0_ref, d0t_ref, wh0_ref, s1_ref, d1t_ref, wh1_ref,
                  wo_ref, ao_ref, who_ref, s3_ref, d3_ref, *, d_hid, n_cls):
    mask = adj_ref[...] > 0.0
    h0 = _elu(_attend(mask, s0_ref[...], d0t_ref[...], wh0_ref[...]))
    h1 = _elu(_attend(mask, s1_ref[...], d1t_ref[...], wh1_ref[...]))
    who = (jnp.dot(h0, wo_ref[:d_hid], preferred_element_type=jnp.float32)
           + jnp.dot(h1, wo_ref[d_hid:], preferred_element_type=jnp.float32))
    who_ref[...] = who
    s3_ref[...] = jnp.dot(who, ao_ref[:n_cls], preferred_element_type=jnp.float32)
    d3_ref[...] = jnp.dot(who, ao_ref[n_cls:], preferred_element_type=jnp.float32)


def _flash12(adj, s0, d0t, wh0, s1, d1t, wh1, wo, ao, br):
    n = adj.shape[0]
    d_hid = wh0.shape[1]
    n_cls = wo.shape[1]
    grid = (n // br,)
    full = lambda shape: pl.BlockSpec(shape, lambda i: (0, 0))
    rowblk = lambda width: pl.BlockSpec((br, width), lambda i: (i, 0))
    body = functools.partial(_flash12_body, d_hid=d_hid, n_cls=n_cls)
    return pl.pallas_call(
        body,
        grid=grid,
        in_specs=[
            rowblk(n),
            rowblk(1),
            full(d0t.shape),
            full(wh0.shape),
            rowblk(1),
            full(d1t.shape),
            full(wh1.shape),
            full(wo.shape), full(ao.shape),
        ],
        out_specs=[rowblk(n_cls), rowblk(1), rowblk(1)],
        out_shape=[
            jax.ShapeDtypeStruct((n, n_cls), jnp.float32),
            jax.ShapeDtypeStruct((n, 1), jnp.float32),
            jax.ShapeDtypeStruct((n, 1), jnp.float32),
        ],
        compiler_params=pltpu.CompilerParams(dimension_semantics=("parallel",)),
        interpret=_INTERPRET,
    )(adj, s0, d0t, wh0, s1, d1t, wh1, wo, ao)


def _flash3_body(adj_ref, s_ref, dt_ref, wh_ref, out_ref):
    mask = adj_ref[...] > 0.0
    out_ref[...] = _elu(_attend(mask, s_ref[...], dt_ref[...], wh_ref[...]))


def _flash3(adj, s3, d3t, who, br):
    n = adj.shape[0]
    n_cls = who.shape[1]
    grid = (n // br,)
    return pl.pallas_call(
        _flash3_body,
        grid=grid,
        in_specs=[
            pl.BlockSpec((br, n), lambda i: (i, 0)),
            pl.BlockSpec((br, 1), lambda i: (i, 0)),
            pl.BlockSpec(d3t.shape, lambda i: (0, 0)),
            pl.BlockSpec(who.shape, lambda i: (0, 0)),
        ],
        out_specs=pl.BlockSpec((br, n_cls), lambda i: (i, 0)),
        out_shape=jax.ShapeDtypeStruct((n, n_cls), jnp.float32),
        compiler_params=pltpu.CompilerParams(dimension_semantics=("parallel",)),
        interpret=_INTERPRET,
    )(adj, s3, d3t, who)


def kernel(x, adj, W0, a0, W1, a1, W_out, a_out):
    n = x.shape[0]
    br = _divisor_block(n, 200)
    wh0, s0, d0, wh1, s1, d1 = _projections(x, W0, a0, W1, a1)
    d0t = jnp.reshape(d0, (1, n))
    d1t = jnp.reshape(d1, (1, n))
    who, s3, d3 = _flash12(adj, s0, d0t, wh0, s1, d1t, wh1, W_out, a_out, br)
    d3t = jnp.reshape(d3, (1, n))
    return _flash3(adj, s3, d3t, who, br)


# speedup vs baseline: 1.1624x; 1.1624x over previous
import functools
import math

import jax
import jax.numpy as jnp
from jax.experimental import pallas as pl
from jax.experimental.pallas import tpu as pltpu

ALPHA = 0.2
LOG2E = math.log2(math.e)
NEG2 = -9e15 * LOG2E

_INTERPRET = False


def _divisor_block(n, target):
    best = None
    for b in range(8, min(n, target) + 1, 8):
        if n % b == 0:
            best = b
    return best if best is not None else n


def _leaky_relu(v):
    return jnp.maximum(v, ALPHA * v)


def _elu(v):
    return jnp.where(v > 0, v, jnp.exp(jnp.minimum(v, 0.0)) - 1.0)


def _attend(mask, s, dt, wh_aug, d_out):
    t = _leaky_relu(s + dt)
    e = jnp.where(mask, t, NEG2)
    m = jnp.max(e, axis=1, keepdims=True)
    p = jnp.exp2(e - m)
    acc = jnp.dot(p, wh_aug, preferred_element_type=jnp.float32)
    return acc[:, :d_out] / acc[:, d_out:d_out + 1]


def _proj_body(x_ref, w0_ref, a0_ref, w1_ref, a1_ref,
               wh0_ref, s0_ref, d0_ref, wh1_ref, s1_ref, d1_ref):
    x = x_ref[...]
    d_hid = w0_ref.shape[1]
    for w_ref, a_ref, wh_ref, s_ref, d_ref in (
        (w0_ref, a0_ref, wh0_ref, s0_ref, d0_ref),
        (w1_ref, a1_ref, wh1_ref, s1_ref, d1_ref),
    ):
        wh = jnp.dot(x, w_ref[...], preferred_element_type=jnp.float32)
        wh_ref[:, :d_hid] = wh
        wh_ref[:, d_hid:] = jnp.ones_like(wh_ref[:, d_hid:])
        s_ref[...] = jnp.dot(wh, a_ref[:d_hid], preferred_element_type=jnp.float32)
        d_ref[...] = jnp.dot(wh, a_ref[d_hid:], preferred_element_type=jnp.float32)


def _projections(x, w0, a0, w1, a1):
    n, nfeat = x.shape
    d_hid = w0.shape[1]
    br = _divisor_block(n, 2000)
    grid = (n // br,)
    out_shapes = []
    for _ in range(2):
        out_shapes += [
            jax.ShapeDtypeStruct((n, d_hid + 1), jnp.float32),
            jax.ShapeDtypeStruct((n, 1), jnp.float32),
            jax.ShapeDtypeStruct((n, 1), jnp.float32),
        ]
    full = lambda shape: pl.BlockSpec(shape, lambda i: (0, 0))
    row = lambda width: pl.BlockSpec((br, width), lambda i: (i, 0))
    return pl.pallas_call(
        _proj_body,
        grid=grid,
        in_specs=[
            row(nfeat),
            full(w0.shape), full(a0.shape),
            full(w1.shape), full(a1.shape),
        ],
        out_specs=[row(d_hid + 1), row(1), row(1)] * 2,
        out_shape=out_shapes,
        compiler_params=pltpu.CompilerParams(dimension_semantics=("parallel",)),
        interpret=_INTERPRET,
    )(x, w0, a0, w1, a1)


def _flash12_body(adj_ref, s0_ref, d0t_ref, wh0_ref, s1_ref, d1t_ref, wh1_ref,
                  wo_ref, ao_ref, who_ref, s3_ref, d3_ref, m8_ref,
                  *, d_hid, n_cls):
    mask = adj_ref[...] > 0.0
    m8_ref[...] = mask[None].astype(jnp.int8)
    h0 = _elu(_attend(mask, s0_ref[...], d0t_ref[...], wh0_ref[...], d_hid))
    h1 = _elu(_attend(mask, s1_ref[...], d1t_ref[...], wh1_ref[...], d_hid))
    who = (jnp.dot(h0, wo_ref[:d_hid], preferred_element_type=jnp.float32)
           + jnp.dot(h1, wo_ref[d_hid:], preferred_element_type=jnp.float32))
    who_ref[:, :n_cls] = who
    who_ref[:, n_cls:] = jnp.ones_like(who_ref[:, n_cls:])
    s3_ref[...] = jnp.dot(who, ao_ref[:n_cls], preferred_element_type=jnp.float32)
    d3_ref[...] = jnp.dot(who, ao_ref[n_cls:], preferred_element_type=jnp.float32)


def _flash12(adj, s0, d0t, wh0, s1, d1t, wh1, wo, ao, br):
    n = adj.shape[0]
    d_hid = wh0.shape[1] - 1
    n_cls = wo.shape[1]
    nb = n // br
    grid = (nb,)
    full = lambda shape: pl.BlockSpec(shape, lambda i: (0, 0))
    rowblk = lambda width: pl.BlockSpec((br, width), lambda i: (i, 0))
    body = functools.partial(_flash12_body, d_hid=d_hid, n_cls=n_cls)
    return pl.pallas_call(
        body,
        grid=grid,
        in_specs=[
            rowblk(n),
            rowblk(1),
            full(d0t.shape),
            full(wh0.shape),
            rowblk(1),
            full(d1t.shape),
            full(wh1.shape),
            full(wo.shape), full(ao.shape),
        ],
        out_specs=[
            rowblk(n_cls + 1), rowblk(1), rowblk(1),
            pl.BlockSpec((1, br, n), lambda i: (i, 0, 0)),
        ],
        out_shape=[
            jax.ShapeDtypeStruct((n, n_cls + 1), jnp.float32),
            jax.ShapeDtypeStruct((n, 1), jnp.float32),
            jax.ShapeDtypeStruct((n, 1), jnp.float32),
            jax.ShapeDtypeStruct((nb, br, n), jnp.int8),
        ],
        compiler_params=pltpu.CompilerParams(dimension_semantics=("parallel",)),
        interpret=_INTERPRET,
    )(adj, s0, d0t, wh0, s1, d1t, wh1, wo, ao)


def _flash3_body(m8_ref, s_ref, dt_ref, wh_ref, out_ref, *, n_cls):
    mask = m8_ref[0] != 0
    out_ref[...] = _elu(
        _attend(mask, s_ref[...], dt_ref[...], wh_ref[...], n_cls))


def _flash3(m8, s3, d3t, who_aug, br):
    nb, _, n = m8.shape
    n_cls = who_aug.shape[1] - 1
    grid = (nb,)
    body = functools.partial(_flash3_body, n_cls=n_cls)
    return pl.pallas_call(
        body,
        grid=grid,
        in_specs=[
            pl.BlockSpec((1, br, n), lambda i: (i, 0, 0)),
            pl.BlockSpec((br, 1), lambda i: (i, 0)),
            pl.BlockSpec(d3t.shape, lambda i: (0, 0)),
            pl.BlockSpec(who_aug.shape, lambda i: (0, 0)),
        ],
        out_specs=pl.BlockSpec((br, n_cls), lambda i: (i, 0)),
        out_shape=jax.ShapeDtypeStruct((n, n_cls), jnp.float32),
        compiler_params=pltpu.CompilerParams(dimension_semantics=("parallel",)),
        interpret=_INTERPRET,
    )(m8, s3, d3t, who_aug)


def kernel(x, adj, W0, a0, W1, a1, W_out, a_out):
    n = x.shape[0]
    br = _divisor_block(n, 200)
    a0s = a0 * LOG2E
    a1s = a1 * LOG2E
    aos = a_out * LOG2E
    wh0, s0, d0, wh1, s1, d1 = _projections(x, W0, a0s, W1, a1s)
    d0t = jnp.reshape(d0, (1, n))
    d1t = jnp.reshape(d1, (1, n))
    who_aug, s3, d3, m8 = _flash12(adj, s0, d0t, wh0, s1, d1t, wh1,
                                   W_out, aos, br)
    d3t = jnp.reshape(d3, (1, n))
    return _flash3(m8, s3, d3t, who_aug, br)


# speedup vs baseline: 1.6288x; 1.4012x over previous
import functools
import math

import jax
import jax.numpy as jnp
from jax.experimental import pallas as pl
from jax.experimental.pallas import tpu as pltpu

ALPHA = 0.2
LOG2E = math.log2(math.e)

_INTERPRET = False


def _divisor_block(n, target):
    best = None
    for b in range(8, min(n, target) + 1, 8):
        if n % b == 0:
            best = b
    return best if best is not None else n


def _elu(v):
    return jnp.where(v > 0, v, jnp.exp(jnp.minimum(v, 0.0)) - 1.0)


def _attend(maskf, r1, r2, c1t, c2t, wh_aug, d_out):
    p = maskf * jnp.maximum(r1 * c1t, r2 * c2t)
    acc = jnp.dot(p, wh_aug, preferred_element_type=jnp.float32)
    num = acc[:, :d_out]
    den = acc[:, d_out:d_out + 1]
    csum = jnp.sum(wh_aug, axis=0, keepdims=True)
    safe = den > 0
    num = jnp.where(safe, num, csum[:, :d_out])
    den = jnp.where(safe, den, csum[:, d_out:d_out + 1])
    return num / den


def _proj_body(x_ref, w0_ref, a0_ref, w1_ref, a1_ref,
               wh0_ref, s0_ref, d0_ref, wh1_ref, s1_ref, d1_ref):
    x = x_ref[...]
    d_hid = w0_ref.shape[1]
    for w_ref, a_ref, wh_ref, s_ref, d_ref in (
        (w0_ref, a0_ref, wh0_ref, s0_ref, d0_ref),
        (w1_ref, a1_ref, wh1_ref, s1_ref, d1_ref),
    ):
        wh = jnp.dot(x, w_ref[...], preferred_element_type=jnp.float32)
        wh_ref[:, :d_hid] = wh
        wh_ref[:, d_hid:] = jnp.ones_like(wh_ref[:, d_hid:])
        s_ref[...] = jnp.dot(wh, a_ref[:d_hid], preferred_element_type=jnp.float32)
        d_ref[...] = jnp.dot(wh, a_ref[d_hid:], preferred_element_type=jnp.float32)


def _projections(x, w0, a0, w1, a1):
    n, nfeat = x.shape
    d_hid = w0.shape[1]
    br = _divisor_block(n, 2000)
    grid = (n // br,)
    out_shapes = []
    for _ in range(2):
        out_shapes += [
            jax.ShapeDtypeStruct((n, d_hid + 1), jnp.float32),
            jax.ShapeDtypeStruct((n, 1), jnp.float32),
            jax.ShapeDtypeStruct((n, 1), jnp.float32),
        ]
    full = lambda shape: pl.BlockSpec(shape, lambda i: (0, 0))
    row = lambda width: pl.BlockSpec((br, width), lambda i: (i, 0))
    return pl.pallas_call(
        _proj_body,
        grid=grid,
        in_specs=[
            row(nfeat),
            full(w0.shape), full(a0.shape),
            full(w1.shape), full(a1.shape),
        ],
        out_specs=[row(d_hid + 1), row(1), row(1)] * 2,
        out_shape=out_shapes,
        compiler_params=pltpu.CompilerParams(dimension_semantics=("parallel",)),
        interpret=_INTERPRET,
    )(x, w0, a0, w1, a1)


def _factor_pair(s, d):
    big = jnp.max(d, axis=1, keepdims=True)
    sd = s + big
    m = jnp.maximum(sd, ALPHA * sd)
    r1 = jnp.exp2((sd - m) * LOG2E)
    r2 = jnp.exp2((ALPHA * sd - m) * LOG2E)
    c1 = jnp.exp2((d - big) * LOG2E)
    c2 = jnp.exp2((ALPHA * (d - big)) * LOG2E)
    return r1, r2, c1, c2


def _factors_body(*refs, n_pairs):
    for k in range(n_pairs):
        s_ref, d_ref = refs[2 * k], refs[2 * k + 1]
        outs = refs[2 * n_pairs + 4 * k: 2 * n_pairs + 4 * k + 4]
        vals = _factor_pair(s_ref[...], d_ref[...])
        for o_ref, v in zip(outs, vals):
            o_ref[...] = v


def _factors(*sd_ts):
    n_pairs = len(sd_ts) // 2
    n = sd_ts[0].shape[1]
    full = pl.BlockSpec((1, n), lambda: (0, 0))
    body = functools.partial(_factors_body, n_pairs=n_pairs)
    return pl.pallas_call(
        body,
        in_specs=[full] * (2 * n_pairs),
        out_specs=[full] * (4 * n_pairs),
        out_shape=[jax.ShapeDtypeStruct((1, n), jnp.float32)] * (4 * n_pairs),
        interpret=_INTERPRET,
    )(*sd_ts)


def _flash12_body(adj_ref, r10_ref, r20_ref, c10_ref, c20_ref, wh0_ref,
                  r11_ref, r21_ref, c11_ref, c21_ref, wh1_ref,
                  wo_ref, ao_ref, who_ref, s3_ref, d3_ref, m8_ref,
                  *, d_hid, n_cls):
    adj = adj_ref[...]
    m8_ref[...] = adj[None].astype(jnp.int8)
    h0 = _elu(_attend(adj, r10_ref[...], r20_ref[...], c10_ref[...],
                      c20_ref[...], wh0_ref[...], d_hid))
    h1 = _elu(_attend(adj, r11_ref[...], r21_ref[...], c11_ref[...],
                      c21_ref[...], wh1_ref[...], d_hid))
    who = (jnp.dot(h0, wo_ref[:d_hid], preferred_element_type=jnp.float32)
           + jnp.dot(h1, wo_ref[d_hid:], preferred_element_type=jnp.float32))
    who_ref[:, :n_cls] = who
    who_ref[:, n_cls:] = jnp.ones_like(who_ref[:, n_cls:])
    s3_ref[...] = jnp.dot(who, ao_ref[:n_cls], preferred_element_type=jnp.float32)
    d3_ref[...] = jnp.dot(who, ao_ref[n_cls:], preferred_element_type=jnp.float32)


def _flash12(adj, h0_vecs, wh0, h1_vecs, wh1, wo, ao, br):
    n = adj.shape[0]
    d_hid = wh0.shape[1] - 1
    n_cls = wo.shape[1]
    nb = n // br
    full = lambda shape: pl.BlockSpec(shape, lambda i: (0, 0))
    rowblk = lambda width: pl.BlockSpec((br, width), lambda i: (i, 0))
    head_specs = [rowblk(1), rowblk(1), full((1, n)), full((1, n))]
    body = functools.partial(_flash12_body, d_hid=d_hid, n_cls=n_cls)
    return pl.pallas_call(
        body,
        grid=(nb,),
        in_specs=(
            [rowblk(n)] + head_specs + [full(wh0.shape)]
            + head_specs + [full(wh1.shape)]
            + [full(wo.shape), full(ao.shape)]
        ),
        out_specs=[
            rowblk(n_cls + 1), rowblk(1), rowblk(1),
            pl.BlockSpec((1, br, n), lambda i: (i, 0, 0)),
        ],
        out_shape=[
            jax.ShapeDtypeStruct((n, n_cls + 1), jnp.float32),
            jax.ShapeDtypeStruct((n, 1), jnp.float32),
            jax.ShapeDtypeStruct((n, 1), jnp.float32),
            jax.ShapeDtypeStruct((nb, br, n), jnp.int8),
        ],
        compiler_params=pltpu.CompilerParams(dimension_semantics=("parallel",)),
        interpret=_INTERPRET,
    )(adj, *h0_vecs, wh0, *h1_vecs, wh1, wo, ao)


def _flash3_body(m8_ref, r1_ref, r2_ref, c1_ref, c2_ref, wh_ref, out_ref,
                 *, n_cls):
    maskf = m8_ref[0].astype(jnp.float32)
    out_ref[...] = _elu(_attend(maskf, r1_ref[...], r2_ref[...], c1_ref[...],
                                c2_ref[...], wh_ref[...], n_cls))


def _flash3(m8, vecs, who_aug, br):
    nb, _, n = m8.shape
    n_cls = who_aug.shape[1] - 1
    full = lambda shape: pl.BlockSpec(shape, lambda i: (0, 0))
    rowblk = lambda width: pl.BlockSpec((br, width), lambda i: (i, 0))
    body = functools.partial(_flash3_body, n_cls=n_cls)
    return pl.pallas_call(
        body,
        grid=(nb,),
        in_specs=[
            pl.BlockSpec((1, br, n), lambda i: (i, 0, 0)),
            rowblk(1), rowblk(1), full((1, n)), full((1, n)),
            full(who_aug.shape),
        ],
        out_specs=rowblk(n_cls),
        out_shape=jax.ShapeDtypeStruct((n, n_cls), jnp.float32),
        compiler_params=pltpu.CompilerParams(dimension_semantics=("parallel",)),
        interpret=_INTERPRET,
    )(m8, *vecs, who_aug)


def kernel(x, adj, W0, a0, W1, a1, W_out, a_out):
    n = x.shape[0]
    br = _divisor_block(n, 200)
    wh0, s0, d0, wh1, s1, d1 = _projections(x, W0, a0, W1, a1)
    row_t = lambda v: jnp.reshape(v, (1, n))
    col = lambda v: jnp.reshape(v, (n, 1))
    r10, r20, c10, c20, r11, r21, c11, c21 = _factors(
        row_t(s0), row_t(d0), row_t(s1), row_t(d1))
    who_aug, s3, d3, m8 = _flash12(
        adj,
        (col(r10), col(r20), c10, c20), wh0,
        (col(r11), col(r21), c11, c21), wh1,
        W_out, a_out, br)
    r13, r23, c13, c23 = _factors(row_t(s3), row_t(d3))
    return _flash3(m8, (col(r13), col(r23), c13, c23), who_aug, br)


# speedup vs baseline: 1.8097x; 1.1111x over previous
import functools
import math

import jax
import jax.numpy as jnp
from jax.experimental import pallas as pl
from jax.experimental.pallas import tpu as pltpu

ALPHA = 0.2
LOG2E = math.log2(math.e)

_INTERPRET = False


def _divisor_block(n, target):
    best = None
    for b in range(8, min(n, target) + 1, 8):
        if n % b == 0:
            best = b
    return best if best is not None else n


def _elu(v):
    return jnp.where(v > 0, v, jnp.exp(jnp.minimum(v, 0.0)) - 1.0)


def _attend(maskf, r1, r2, c1t, c2t, wh_aug, csum, d_out):
    p = maskf * jnp.maximum(r1 * c1t, r2 * c2t)
    acc = jnp.dot(p, wh_aug, preferred_element_type=jnp.float32)
    num = acc[:, :d_out]
    den = acc[:, d_out:d_out + 1]
    safe = den > 0
    num = jnp.where(safe, num, csum[:, :d_out])
    den = jnp.where(safe, den, csum[:, d_out:d_out + 1])
    return num / den


def _proj_body(x_ref, w0_ref, a0_ref, w1_ref, a1_ref,
               wh0_ref, s0_ref, d0_ref, wh1_ref, s1_ref, d1_ref):
    x = x_ref[...]
    d_hid = w0_ref.shape[1]
    for w_ref, a_ref, wh_ref, s_ref, d_ref in (
        (w0_ref, a0_ref, wh0_ref, s0_ref, d0_ref),
        (w1_ref, a1_ref, wh1_ref, s1_ref, d1_ref),
    ):
        wh = jnp.dot(x, w_ref[...], preferred_element_type=jnp.float32)
        wh_ref[:, :d_hid] = wh
        wh_ref[:, d_hid:] = jnp.ones_like(wh_ref[:, d_hid:])
        s_ref[...] = jnp.dot(wh, a_ref[:d_hid], preferred_element_type=jnp.float32)
        d_ref[...] = jnp.dot(wh, a_ref[d_hid:], preferred_element_type=jnp.float32)


def _projections(x, w0, a0, w1, a1):
    n, nfeat = x.shape
    d_hid = w0.shape[1]
    br = _divisor_block(n, 2000)
    grid = (n // br,)
    out_shapes = []
    for _ in range(2):
        out_shapes += [
            jax.ShapeDtypeStruct((n, d_hid + 1), jnp.float32),
            jax.ShapeDtypeStruct((n, 1), jnp.float32),
            jax.ShapeDtypeStruct((n, 1), jnp.float32),
        ]
    full = lambda shape: pl.BlockSpec(shape, lambda i: (0, 0))
    row = lambda width: pl.BlockSpec((br, width), lambda i: (i, 0))
    return pl.pallas_call(
        _proj_body,
        grid=grid,
        in_specs=[
            row(nfeat),
            full(w0.shape), full(a0.shape),
            full(w1.shape), full(a1.shape),
        ],
        out_specs=[row(d_hid + 1), row(1), row(1)] * 2,
        out_shape=out_shapes,
        compiler_params=pltpu.CompilerParams(dimension_semantics=("parallel",)),
        interpret=_INTERPRET,
    )(x, w0, a0, w1, a1)


def _factor_pair(s, d):
    big = jnp.max(d, axis=1, keepdims=True)
    sd = s + big
    m = jnp.maximum(sd, ALPHA * sd)
    r1 = jnp.exp2((sd - m) * LOG2E)
    r2 = jnp.exp2((ALPHA * sd - m) * LOG2E)
    c1 = jnp.exp2((d - big) * LOG2E)
    c2 = jnp.exp2((ALPHA * (d - big)) * LOG2E)
    return r1, r2, c1, c2


def _factors_body(*refs, n_pairs):
    ins = refs[:3 * n_pairs]
    outs = refs[3 * n_pairs:]
    for k in range(n_pairs):
        s_ref, d_ref, wh_ref = ins[3 * k], ins[3 * k + 1], ins[3 * k + 2]
        o = outs[5 * k:5 * k + 5]
        vals = _factor_pair(s_ref[...], d_ref[...])
        for o_ref, v in zip(o[:4], vals):
            o_ref[...] = v
        o[4][...] = jnp.sum(wh_ref[...], axis=0, keepdims=True)


def _factors(*triples):
    n_pairs = len(triples) // 3
    n = triples[0].shape[1]
    vec = pl.BlockSpec((1, n), lambda: (0, 0))
    in_specs, out_specs, out_shape = [], [], []
    for k in range(n_pairs):
        wh = triples[3 * k + 2]
        in_specs += [vec, vec, pl.BlockSpec(wh.shape, lambda: (0, 0))]
        out_specs += [vec] * 4 + [pl.BlockSpec((1, wh.shape[1]), lambda: (0, 0))]
        out_shape += [jax.ShapeDtypeStruct((1, n), jnp.float32)] * 4
        out_shape += [jax.ShapeDtypeStruct((1, wh.shape[1]), jnp.float32)]
    body = functools.partial(_factors_body, n_pairs=n_pairs)
    return pl.pallas_call(
        body,
        in_specs=in_specs,
        out_specs=out_specs,
        out_shape=out_shape,
        interpret=_INTERPRET,
    )(*triples)


def _flash12_body(adj_ref, r10_ref, r20_ref, c10_ref, c20_ref, wh0_ref,
                  cs0_ref, r11_ref, r21_ref, c11_ref, c21_ref, wh1_ref,
                  cs1_ref, wo_ref, ao_ref, who_ref, s3_ref, d3_ref, m8_ref,
                  *, d_hid, n_cls):
    adj = adj_ref[...]
    m8_ref[...] = adj[None].astype(jnp.int8)
    h0 = _elu(_attend(adj, r10_ref[...], r20_ref[...], c10_ref[...],
                      c20_ref[...], wh0_ref[...], cs0_ref[...], d_hid))
    h1 = _elu(_attend(adj, r11_ref[...], r21_ref[...], c11_ref[...],
                      c21_ref[...], wh1_ref[...], cs1_ref[...], d_hid))
    who = (jnp.dot(h0, wo_ref[:d_hid], preferred_element_type=jnp.float32)
           + jnp.dot(h1, wo_ref[d_hid:], preferred_element_type=jnp.float32))
    who_ref[:, :n_cls] = who
    who_ref[:, n_cls:] = jnp.ones_like(who_ref[:, n_cls:])
    s3_ref[...] = jnp.dot(who, ao_ref[:n_cls], preferred_element_type=jnp.float32)
    d3_ref[...] = jnp.dot(who, ao_ref[n_cls:], preferred_element_type=jnp.float32)


def _flash12(adj, h0_vecs, wh0, cs0, h1_vecs, wh1, cs1, wo, ao, br):
    n = adj.shape[0]
    d_hid = wh0.shape[1] - 1
    n_cls = wo.shape[1]
    nb = n // br
    full = lambda shape: pl.BlockSpec(shape, lambda i: (0, 0))
    rowblk = lambda width: pl.BlockSpec((br, width), lambda i: (i, 0))
    head_specs = [rowblk(1), rowblk(1), full((1, n)), full((1, n))]
    body = functools.partial(_flash12_body, d_hid=d_hid, n_cls=n_cls)
    return pl.pallas_call(
        body,
        grid=(nb,),
        in_specs=(
            [rowblk(n)] + head_specs + [full(wh0.shape), full((1, d_hid + 1))]
            + head_specs + [full(wh1.shape), full((1, d_hid + 1))]
            + [full(wo.shape), full(ao.shape)]
        ),
        out_specs=[
            rowblk(n_cls + 1), rowblk(1), rowblk(1),
            pl.BlockSpec((1, br, n), lambda i: (i, 0, 0)),
        ],
        out_shape=[
            jax.ShapeDtypeStruct((n, n_cls + 1), jnp.float32),
            jax.ShapeDtypeStruct((n, 1), jnp.float32),
            jax.ShapeDtypeStruct((n, 1), jnp.float32),
            jax.ShapeDtypeStruct((nb, br, n), jnp.int8),
        ],
        compiler_params=pltpu.CompilerParams(dimension_semantics=("parallel",)),
        interpret=_INTERPRET,
    )(adj, *h0_vecs, wh0, cs0, *h1_vecs, wh1, cs1, wo, ao)


def _flash3_body(m8_ref, r1_ref, r2_ref, c1_ref, c2_ref, wh_ref, cs_ref,
                 out_ref, *, n_cls):
    maskf = m8_ref[0].astype(jnp.float32)
    out_ref[...] = _elu(_attend(maskf, r1_ref[...], r2_ref[...], c1_ref[...],
                                c2_ref[...], wh_ref[...], cs_ref[...], n_cls))


def _flash3(m8, vecs, who_aug, cs3, br):
    nb, _, n = m8.shape
    n_cls = who_aug.shape[1] - 1
    full = lambda shape: pl.BlockSpec(shape, lambda i: (0, 0))
    rowblk = lambda width: pl.BlockSpec((br, width), lambda i: (i, 0))
    body = functools.partial(_flash3_body, n_cls=n_cls)
    return pl.pallas_call(
        body,
        grid=(nb,),
        in_specs=[
            pl.BlockSpec((1, br, n), lambda i: (i, 0, 0)),
            rowblk(1), rowblk(1), full((1, n)), full((1, n)),
            full(who_aug.shape), full((1, who_aug.shape[1])),
        ],
        out_specs=rowblk(n_cls),
        out_shape=jax.ShapeDtypeStruct((n, n_cls), jnp.float32),
        compiler_params=pltpu.CompilerParams(dimension_semantics=("parallel",)),
        interpret=_INTERPRET,
    )(m8, *vecs, who_aug, cs3)


def kernel(x, adj, W0, a0, W1, a1, W_out, a_out):
    n = x.shape[0]
    br = _divisor_block(n, 200)
    wh0, s0, d0, wh1, s1, d1 = _projections(x, W0, a0, W1, a1)
    row_t = lambda v: jnp.reshape(v, (1, n))
    col = lambda v: jnp.reshape(v, (n, 1))
    (r10, r20, c10, c20, cs0,
     r11, r21, c11, c21, cs1) = _factors(
        row_t(s0), row_t(d0), wh0, row_t(s1), row_t(d1), wh1)
    who_aug, s3, d3, m8 = _flash12(
        adj,
        (col(r10), col(r20), c10, c20), wh0, cs0,
        (col(r11), col(r21), c11, c21), wh1, cs1,
        W_out, a_out, br)
    r13, r23, c13, c23, cs3 = _factors(row_t(s3), row_t(d3), who_aug)
    return _flash3(m8, (col(r13), col(r23), c13, c23), who_aug, cs3, br)


# speedup vs baseline: 1.9834x; 1.0960x over previous
import functools
import math

import jax
import jax.numpy as jnp
from jax.experimental import pallas as pl
from jax.experimental.pallas import tpu as pltpu

ALPHA = 0.2
LOG2E = math.log2(math.e)

_INTERPRET = False


def _divisor_block(n, target):
    best = None
    for b in range(8, min(n, target) + 1, 8):
        if n % b == 0:
            best = b
    return best if best is not None else n


def _elu(v):
    return jnp.where(v > 0, v, jnp.exp(jnp.minimum(v, 0.0)) - 1.0)


def _col_factors(dt, c1_ref, c2_ref, dmax_ref):
    big = jnp.max(dt, axis=1, keepdims=True)
    dmax_ref[...] = big
    c1_ref[...] = jnp.exp2((dt - big) * LOG2E)
    c2_ref[...] = jnp.exp2((ALPHA * (dt - big)) * LOG2E)


def _row_factors(s, dmax):
    sd = s + dmax
    m = jnp.maximum(sd, ALPHA * sd)
    r1 = jnp.exp2((sd - m) * LOG2E)
    r2 = jnp.exp2((ALPHA * sd - m) * LOG2E)
    return r1, r2


def _attend(maskf, r1, r2, c1t, c2t, wh_aug, csum, d_out):
    p = maskf * jnp.maximum(r1 * c1t, r2 * c2t)
    acc = jnp.dot(p, wh_aug, preferred_element_type=jnp.float32)
    num = acc[:, :d_out]
    den = acc[:, d_out:d_out + 1]
    safe = den > 0
    num = jnp.where(safe, num, csum[:, :d_out])
    den = jnp.where(safe, den, csum[:, d_out:d_out + 1])
    return num / den


def _proj_body(x_ref, w0_ref, a0_ref, w1_ref, a1_ref,
               wh0_ref, s0_ref, d0_ref, wh1_ref, s1_ref, d1_ref):
    x = x_ref[...]
    d_hid = w0_ref.shape[1]
    for w_ref, a_ref, wh_ref, s_ref, d_ref in (
        (w0_ref, a0_ref, wh0_ref, s0_ref, d0_ref),
        (w1_ref, a1_ref, wh1_ref, s1_ref, d1_ref),
    ):
        wh = jnp.dot(x, w_ref[...], preferred_element_type=jnp.float32)
        wh_ref[:, :d_hid] = wh
        wh_ref[:, d_hid:] = jnp.ones_like(wh_ref[:, d_hid:])
        s_ref[...] = jnp.dot(wh, a_ref[:d_hid], preferred_element_type=jnp.float32)
        d_ref[...] = jnp.dot(wh, a_ref[d_hid:], preferred_element_type=jnp.float32)


def _projections(x, w0, a0, w1, a1):
    n, nfeat = x.shape
    d_hid = w0.shape[1]
    br = _divisor_block(n, 2000)
    grid = (n // br,)
    out_shapes = []
    for _ in range(2):
        out_shapes += [
            jax.ShapeDtypeStruct((n, d_hid + 1), jnp.float32),
            jax.ShapeDtypeStruct((n, 1), jnp.float32),
            jax.ShapeDtypeStruct((n, 1), jnp.float32),
        ]
    full = lambda shape: pl.BlockSpec(shape, lambda i: (0, 0))
    row = lambda width: pl.BlockSpec((br, width), lambda i: (i, 0))
    return pl.pallas_call(
        _proj_body,
        grid=grid,
        in_specs=[
            row(nfeat),
            full(w0.shape), full(a0.shape),
            full(w1.shape), full(a1.shape),
        ],
        out_specs=[row(d_hid + 1), row(1), row(1)] * 2,
        out_shape=out_shapes,
        compiler_params=pltpu.CompilerParams(dimension_semantics=("parallel",)),
        interpret=_INTERPRET,
    )(x, w0, a0, w1, a1)


def _flash12_body(adj_ref, s0_ref, d0t_ref, wh0_ref, s1_ref, d1t_ref, wh1_ref,
                  wo_ref, ao_ref,
                  who_ref, s3_ref, d3_ref, m8_ref,
                  c10, c20, dm0, cs0, c11, c21, dm1, cs1,
                  *, d_hid, n_cls):
    i = pl.program_id(0)

    @pl.when(i == 0)
    def _init():
        _col_factors(d0t_ref[...], c10, c20, dm0)
        _col_factors(d1t_ref[...], c11, c21, dm1)
        cs0[...] = jnp.sum(wh0_ref[...], axis=0, keepdims=True)
        cs1[...] = jnp.sum(wh1_ref[...], axis=0, keepdims=True)

    adj = adj_ref[...]
    m8_ref[...] = adj[None].astype(jnp.int8)
    hs = []
    for s_ref, wh_ref, c1, c2, dm, cs in (
        (s0_ref, wh0_ref, c10, c20, dm0, cs0),
        (s1_ref, wh1_ref, c11, c21, dm1, cs1),
    ):
        r1, r2 = _row_factors(s_ref[...], dm[...])
        hs.append(_elu(_attend(adj, r1, r2, c1[...], c2[...],
                               wh_ref[...], cs[...], d_hid)))
    who = (jnp.dot(hs[0], wo_ref[:d_hid], preferred_element_type=jnp.float32)
           + jnp.dot(hs[1], wo_ref[d_hid:], preferred_element_type=jnp.float32))
    who_ref[:, :n_cls] = who
    who_ref[:, n_cls:] = jnp.ones_like(who_ref[:, n_cls:])
    s3_ref[...] = jnp.dot(who, ao_ref[:n_cls], preferred_element_type=jnp.float32)
    d3_ref[...] = jnp.dot(who, ao_ref[n_cls:], preferred_element_type=jnp.float32)


def _flash12(adj, s0, d0t, wh0, s1, d1t, wh1, wo, ao, br):
    n = adj.shape[0]
    d_hid = wh0.shape[1] - 1
    n_cls = wo.shape[1]
    nb = n // br
    full = lambda shape: pl.BlockSpec(shape, lambda i: (0, 0))
    rowblk = lambda width: pl.BlockSpec((br, width), lambda i: (i, 0))
    vec = pltpu.VMEM((1, n), jnp.float32)
    scal = pltpu.VMEM((1, 1), jnp.float32)
    csum = pltpu.VMEM((1, d_hid + 1), jnp.float32)
    body = functools.partial(_flash12_body, d_hid=d_hid, n_cls=n_cls)
    return pl.pallas_call(
        body,
        grid=(nb,),
        in_specs=[
            rowblk(n),
            rowblk(1), full(d0t.shape), full(wh0.shape),
            rowblk(1), full(d1t.shape), full(wh1.shape),
            full(wo.shape), full(ao.shape),
        ],
        out_specs=[
            rowblk(n_cls + 1), rowblk(1), rowblk(1),
            pl.BlockSpec((1, br, n), lambda i: (i, 0, 0)),
        ],
        out_shape=[
            jax.ShapeDtypeStruct((n, n_cls + 1), jnp.float32),
            jax.ShapeDtypeStruct((n, 1), jnp.float32),
            jax.ShapeDtypeStruct((n, 1), jnp.float32),
            jax.ShapeDtypeStruct((nb, br, n), jnp.int8),
        ],
        scratch_shapes=[vec, vec, scal, csum, vec, vec, scal, csum],
        interpret=_INTERPRET,
    )(adj, s0, d0t, wh0, s1, d1t, wh1, wo, ao)


def _flash3_body(m8_ref, s_ref, dt_ref, wh_ref, out_ref,
                 c1, c2, dm, cs, *, n_cls):
    i = pl.program_id(0)

    @pl.when(i == 0)
    def _init():
        _col_factors(dt_ref[...], c1, c2, dm)
        cs[...] = jnp.sum(wh_ref[...], axis=0, keepdims=True)

    maskf = m8_ref[0].astype(jnp.float32)
    r1, r2 = _row_factors(s_ref[...], dm[...])
    out_ref[...] = _elu(_attend(maskf, r1, r2, c1[...], c2[...],
                                wh_ref[...], cs[...], n_cls))


def _flash3(m8, s3, d3t, who_aug, br):
    nb, _, n = m8.shape
    n_cls = who_aug.shape[1] - 1
    full = lambda shape: pl.BlockSpec(shape, lambda i: (0, 0))
    rowblk = lambda width: pl.BlockSpec((br, width), lambda i: (i, 0))
    body = functools.partial(_flash3_body, n_cls=n_cls)
    return pl.pallas_call(
        body,
        grid=(nb,),
        in_specs=[
            pl.BlockSpec((1, br, n), lambda i: (i, 0, 0)),
            rowblk(1), full(d3t.shape), full(who_aug.shape),
        ],
        out_specs=rowblk(n_cls),
        out_shape=jax.ShapeDtypeStruct((n, n_cls), jnp.float32),
        scratch_shapes=[
            pltpu.VMEM((1, n), jnp.float32), pltpu.VMEM((1, n), jnp.float32),
            pltpu.VMEM((1, 1), jnp.float32),
            pltpu.VMEM((1, n_cls + 1), jnp.float32),
        ],
        interpret=_INTERPRET,
    )(m8, s3, d3t, who_aug)


def kernel(x, adj, W0, a0, W1, a1, W_out, a_out):
    n = x.shape[0]
    br = _divisor_block(n, 200)
    wh0, s0, d0, wh1, s1, d1 = _projections(x, W0, a0, W1, a1)
    d0t = jnp.reshape(d0, (1, n))
    d1t = jnp.reshape(d1, (1, n))
    who_aug, s3, d3, m8 = _flash12(adj, s0, d0t, wh0, s1, d1t, wh1,
                                   W_out, a_out, br)
    d3t = jnp.reshape(d3, (1, n))
    return _flash3(m8, s3, d3t, who_aug, br)
